# baseline (speedup 1.0000x reference)
import functools

import jax
import jax.numpy as jnp
from jax import lax
from jax.experimental import pallas as pl
from jax.experimental.pallas import tpu as pltpu
from jax.experimental.pallas import tpu_sc as plsc

N = 10000
E = 160000
D = 256
HD = 128
NC = 2
NS = 16
EPT = E // NS
EPT_PAD = 10240
K = 128
PHASES = 2
CPP = EPT_PAD // K // PHASES
NPAD = 10240
RPT = NPAD // NS
ZROWS = 32
DEG_PAD = 10240
DPT = DEG_PAD // NS


def _sc_agg_body(x_hbm, src_hbm, dst_hbm, summed_hbm, deg_hbm,
                 src_v, dst_v, rows0, rows1, ones_v, zbuf, zdeg, acc, degacc,
                 sem0, sem1):
    c = lax.axis_index("c")
    s = lax.axis_index("s")
    zero16 = jnp.zeros((16,), jnp.float32)
    one16 = jnp.ones((16,), jnp.float32)

    def _zrow(t, carry):
        i = t // (HD // 16)
        k = t % (HD // 16)
        zbuf[i, pl.ds(k * 16, 16)] = zero16
        return carry
    lax.fori_loop(0, ZROWS * (HD // 16), _zrow, None)

    def _zdeg(t, carry):
        zdeg[pl.ds(t * 16, 16)] = zero16
        return carry
    lax.fori_loop(0, DPT // 16, _zdeg, None)

    def _ones(t, carry):
        ones_v[pl.ds(t * 16, 16)] = one16
        return carry
    lax.fori_loop(0, K // 16, _ones, None)

    for k in range(RPT // ZROWS):
        pltpu.sync_copy(zbuf, acc.at[pl.ds(s * RPT + k * ZROWS, ZROWS)])

    @pl.when(c == 0)
    def _():
        pltpu.sync_copy(zdeg, degacc.at[pl.ds(s * DPT, DPT)])

    plsc.subcore_barrier()

    def _start_gather(j, buf, sem):
        pltpu.async_copy(x_hbm.at[src_v.at[j]], buf, sem)

    def _finish(j, buf, sem):
        pltpu.make_async_copy(x_hbm.at[src_v.at[j]], buf, sem).wait()

    for p in range(PHASES):
        pltpu.sync_copy(src_hbm.at[c, s, p], src_v)
        pltpu.sync_copy(dst_hbm.at[s, p], dst_v)
        _start_gather(0, rows0, sem0)

        def _pair(t, carry):
            j0 = 2 * t
            _start_gather(j0 + 1, rows1, sem1)
            _finish(j0, rows0, sem0)

            @pl.when(j0 + 2 < CPP)
            def _():
                _start_gather(j0 + 2, rows0, sem0)
            _finish(j0 + 1, rows1, sem1)
            return carry
        lax.fori_loop(0, CPP // 2, _pair, None)

    plsc.subcore_barrier()

    pltpu.sync_copy(acc.at[pl.ds(s * RPT, RPT)], summed_hbm.at[c, pl.ds(s * RPT, RPT)])

    @pl.when(c == 0)
    def _():
        pltpu.sync_copy(degacc.at[pl.ds(s * DPT, DPT)], deg_hbm.at[pl.ds(s * DPT, DPT)])


_sc_aggregate = functools.partial(
    pl.kernel,
    out_type=[jax.ShapeDtypeStruct((NC, NPAD, HD), jnp.float32),
              jax.ShapeDtypeStruct((DEG_PAD,), jnp.float32)],
    mesh=plsc.VectorSubcoreMesh(core_axis_name="c", subcore_axis_name="s"),
    scratch_types=[
        pltpu.VMEM((CPP, K), jnp.int32),
        pltpu.VMEM((CPP, K), jnp.int32),
        pltpu.VMEM((K, HD), jnp.float32),
        pltpu.VMEM((K, HD), jnp.float32),
        pltpu.VMEM((K,), jnp.float32),
        pltpu.VMEM((ZROWS, HD), jnp.float32),
        pltpu.VMEM((DPT,), jnp.float32),
        pltpu.VMEM_SHARED((NPAD, HD), jnp.float32),
        pltpu.VMEM_SHARED((DEG_PAD,), jnp.float32),
        pltpu.SemaphoreType.DMA,
        pltpu.SemaphoreType.DMA,
    ],
)(_sc_agg_body)


def _tc_layer_body(relu, in_split, out_split,
                   sref, dref, xref, wlref, bref, wrref, oref):
    agg = jnp.concatenate([sref[0], sref[1]], axis=-1)
    rec = 1.0 / jnp.maximum(dref[...], 1.0)
    agg = agg * rec
    if in_split:
        xx = jnp.concatenate([xref[0], xref[1]], axis=-1)
    else:
        xx = xref[...]
    o = (jnp.dot(agg, wlref[...], preferred_element_type=jnp.float32)
         + bref[...]
         + jnp.dot(xx, wrref[...], preferred_element_type=jnp.float32))
    if relu:
        o = jnp.maximum(o, 0.0)
    if out_split:
        oref[0] = o[:, :HD]
        oref[1] = o[:, HD:]
    else:
        oref[...] = o


def _tc_layer(summed, deg_col, xin, W_l, b, W_r, *, relu, in_split, out_split):
    BM = 1000
    grid = (N // BM,)
    split_spec = pl.BlockSpec((NC, BM, HD), lambda i: (0, i, 0))
    dense_spec = pl.BlockSpec((BM, D), lambda i: (i, 0))
    in_specs = [
        split_spec,
        pl.BlockSpec((BM, 1), lambda i: (i, 0)),
        split_spec if in_split else dense_spec,
        pl.BlockSpec((D, D), lambda i: (0, 0)),
        pl.BlockSpec((1, D), lambda i: (0, 0)),
        pl.BlockSpec((D, D), lambda i: (0, 0)),
    ]
    if out_split:
        out_spec = split_spec
        out_shape = jax.ShapeDtypeStruct((NC, N, HD), jnp.float32)
    else:
        out_spec = dense_spec
        out_shape = jax.ShapeDtypeStruct((N, D), jnp.float32)
    return pl.pallas_call(
        functools.partial(_tc_layer_body, relu, in_split, out_split),
        grid=grid,
        in_specs=in_specs,
        out_specs=out_spec,
        out_shape=out_shape,
    )(summed, deg_col, xin, W_l, b.reshape(1, D), W_r)


def kernel(x, edge_index, W1_l, b1, W1_r, W2_l, b2, W2_r):
    src = jnp.sort(edge_index[0].astype(jnp.int32))
    dst = edge_index[1].astype(jnp.int32)
    npad = EPT_PAD - EPT
    sr = jnp.concatenate(
        [src.reshape(NS, EPT), jnp.zeros((NS, npad), jnp.int32)], axis=1
    ).reshape(NS, PHASES, CPP, K)
    src_idx = jnp.stack([sr, sr + N])
    pad_rows = jnp.broadcast_to(N + jnp.arange(npad, dtype=jnp.int32),
                                (NS, npad))
    dst_idx = jnp.concatenate(
        [dst.reshape(NS, EPT), pad_rows], axis=1
    ).reshape(NS, PHASES, CPP, K)
    x_flat = jnp.concatenate([x[:, :HD], x[:, HD:]], axis=0)

    summed1, deg_pad = _sc_aggregate(x_flat, src_idx, dst_idx)
    deg_col = deg_pad[:N].reshape(N, 1)
    h_split = _tc_layer(summed1, deg_col, x, W1_l, b1, W1_r,
                        relu=True, in_split=False, out_split=True)
    summed2, _ = _sc_aggregate(h_split.reshape(NC * N, HD), src_idx, dst_idx)
    out = _tc_layer(summed2, deg_col, h_split, W2_l, b2, W2_r,
                    relu=False, in_split=True, out_split=False)
    return out


# speedup vs baseline: 1.3911x; 1.3911x over previous
import functools

import jax
import jax.numpy as jnp
from jax import lax
from jax.experimental import pallas as pl
from jax.experimental.pallas import tpu as pltpu
from jax.experimental.pallas import tpu_sc as plsc

N = 10000
E = 160000
D = 256
HD = 128
NC = 2
NS = 16
EPT = E // NS
EPT_PAD = 10240
K = 128
NCHUNK = EPT_PAD // K
NPAD = 10240
RPT = NPAD // NS
ZROWS = 32
DPT = NPAD // NS


def _sc_agg_body(with_deg, x_hbm, src_hbm, dst_hbm, summed_hbm, deg_hbm,
                 src_v, dst_v, rows_v, ones_v, zbuf, zdeg, acc, degacc, sem):
    c = lax.axis_index("c")
    s = lax.axis_index("s")
    zero16 = jnp.zeros((16,), jnp.float32)
    one16 = jnp.ones((16,), jnp.float32)

    def _zrow(t, carry):
        i = t // (HD // 16)
        k = t % (HD // 16)
        zbuf[i, pl.ds(k * 16, 16)] = zero16
        return carry
    lax.fori_loop(0, ZROWS * (HD // 16), _zrow, None)

    def _zdeg(t, carry):
        zdeg[pl.ds(t * 16, 16)] = zero16
        return carry
    lax.fori_loop(0, DPT // 16, _zdeg, None)

    def _ones(t, carry):
        ones_v[pl.ds(t * 16, 16)] = one16
        return carry
    lax.fori_loop(0, K // 16, _ones, None)

    for k in range(RPT // ZROWS):
        pltpu.sync_copy(zbuf, acc.at[pl.ds(s * RPT + k * ZROWS, ZROWS)])

    if with_deg:
        @pl.when(c == 0)
        def _():
            pltpu.sync_copy(zdeg, degacc.at[pl.ds(s * DPT, DPT)])

    plsc.subcore_barrier()

    pltpu.sync_copy(src_hbm.at[c, s], src_v)
    pltpu.sync_copy(dst_hbm.at[s], dst_v)

    def _chunk(j, carry):
        pltpu.async_copy(x_hbm.at[src_v.at[j]], rows_v, sem).wait()
        pltpu.sync_copy(rows_v, acc.at[dst_v.at[j]], add=True)

        if with_deg:
            @pl.when(c == 0)
            def _():
                pltpu.sync_copy(ones_v, degacc.at[dst_v.at[j]], add=True)
        return carry
    lax.fori_loop(0, NCHUNK, _chunk, None)

    plsc.subcore_barrier()

    pltpu.sync_copy(acc.at[pl.ds(s * RPT, RPT)],
                    summed_hbm.at[c, pl.ds(s * RPT, RPT)])

    if with_deg:
        @pl.when(c == 0)
        def _():
            pltpu.sync_copy(degacc.at[pl.ds(s * DPT, DPT)],
                            deg_hbm.at[pl.ds(s * DPT, DPT)])


def _make_sc_aggregate(with_deg):
    return functools.partial(
        pl.kernel,
        out_type=[jax.ShapeDtypeStruct((NC, NPAD, HD), jnp.float32),
                  jax.ShapeDtypeStruct((NPAD,), jnp.float32)],
        mesh=plsc.VectorSubcoreMesh(core_axis_name="c", subcore_axis_name="s"),
        scratch_types=[
            pltpu.VMEM((NCHUNK, K), jnp.int32),
            pltpu.VMEM((NCHUNK, K), jnp.int32),
            pltpu.VMEM((K, HD), jnp.float32),
            pltpu.VMEM((K,), jnp.float32),
            pltpu.VMEM((ZROWS, HD), jnp.float32),
            pltpu.VMEM((DPT,), jnp.float32),
            pltpu.VMEM_SHARED((NPAD, HD), jnp.float32),
            pltpu.VMEM_SHARED((NPAD,), jnp.float32),
            pltpu.SemaphoreType.DMA,
        ],
    )(functools.partial(_sc_agg_body, with_deg))


_sc_aggregate_deg = _make_sc_aggregate(True)
_sc_aggregate = _make_sc_aggregate(False)


def _tc_layer_body(relu, in_split, out_split,
                   sref, dref, xref, wlref, bref, wrref, oref):
    agg = jnp.concatenate([sref[0], sref[1]], axis=-1)
    rec = 1.0 / jnp.maximum(dref[...], 1.0)
    agg = agg * rec
    if in_split:
        xx = jnp.concatenate([xref[0], xref[1]], axis=-1)
    else:
        xx = xref[...]
    o = (jnp.dot(agg, wlref[...], preferred_element_type=jnp.float32)
         + bref[...]
         + jnp.dot(xx, wrref[...], preferred_element_type=jnp.float32))
    if relu:
        o = jnp.maximum(o, 0.0)
    if out_split:
        oref[0] = o[:, :HD]
        oref[1] = o[:, HD:]
    else:
        oref[...] = o


def _tc_layer(summed, deg_col, xin, W_l, b, W_r, *, relu, in_split, out_split):
    BM = 1000
    grid = (N // BM,)
    split_spec = pl.BlockSpec((NC, BM, HD), lambda i: (0, i, 0))
    dense_spec = pl.BlockSpec((BM, D), lambda i: (i, 0))
    in_specs = [
        split_spec,
        pl.BlockSpec((BM, 1), lambda i: (i, 0)),
        split_spec if in_split else dense_spec,
        pl.BlockSpec((D, D), lambda i: (0, 0)),
        pl.BlockSpec((1, D), lambda i: (0, 0)),
        pl.BlockSpec((D, D), lambda i: (0, 0)),
    ]
    if out_split:
        out_spec = split_spec
        out_shape = jax.ShapeDtypeStruct((NC, N, HD), jnp.float32)
    else:
        out_spec = dense_spec
        out_shape = jax.ShapeDtypeStruct((N, D), jnp.float32)
    return pl.pallas_call(
        functools.partial(_tc_layer_body, relu, in_split, out_split),
        grid=grid,
        in_specs=in_specs,
        out_specs=out_spec,
        out_shape=out_shape,
    )(summed, deg_col, xin, W_l, b.reshape(1, D), W_r)


def kernel(x, edge_index, W1_l, b1, W1_r, W2_l, b2, W2_r):
    src = edge_index[0].astype(jnp.int32)
    dst = edge_index[1].astype(jnp.int32)
    npad = EPT_PAD - EPT
    sr = jnp.concatenate(
        [src.reshape(NS, EPT), jnp.zeros((NS, npad), jnp.int32)], axis=1
    ).reshape(NS, NCHUNK, K)
    src_idx = jnp.stack([sr, sr + N])
    pad_rows = jnp.broadcast_to(N + jnp.arange(npad, dtype=jnp.int32),
                                (NS, npad))
    dst_idx = jnp.concatenate(
        [dst.reshape(NS, EPT), pad_rows], axis=1
    ).reshape(NS, NCHUNK, K)
    x_flat = jnp.concatenate([x[:, :HD], x[:, HD:]], axis=0)

    summed1, deg_pad = _sc_aggregate_deg(x_flat, src_idx, dst_idx)
    deg_col = deg_pad[:N].reshape(N, 1)
    h_split = _tc_layer(summed1, deg_col, x, W1_l, b1, W1_r,
                        relu=True, in_split=False, out_split=True)
    summed2, _ = _sc_aggregate(h_split.reshape(NC * N, HD), src_idx, dst_idx)
    out = _tc_layer(summed2, deg_col, h_split, W2_l, b2, W2_r,
                    relu=False, in_split=True, out_split=False)
    return out


# speedup vs baseline: 3.0165x; 2.1684x over previous
import functools

import jax
import jax.numpy as jnp
from jax import lax
from jax.experimental import pallas as pl
from jax.experimental.pallas import tpu as pltpu
from jax.experimental.pallas import tpu_sc as plsc

N = 10000
E = 160000
D = 256
HD = 128
NC = 2
NS = 16
EPT = E // NS
K = 80
NCHUNK = EPT // K
PHASES = 5
CPP = NCHUNK // PHASES
NPAD = 10240
RPT = NPAD // NS
ZROWS = 32
DPT = NPAD // NS


def _sc_agg_body(with_deg, x_hbm, src_hbm, dst_hbm, summed_hbm, deg_hbm,
                 src0, src1, dst0, dst1, rows0, rows1, ones_v, zdeg,
                 acc, degacc, sem0, sem1, semi):
    c = lax.axis_index("c")
    s = lax.axis_index("s")
    zero16 = jnp.zeros((16,), jnp.float32)
    one16 = jnp.ones((16,), jnp.float32)

    def _zrow(t, carry):
        i = t // (HD // 16)
        k = t % (HD // 16)
        rows0[i, pl.ds(k * 16, 16)] = zero16
        return carry
    lax.fori_loop(0, K * (HD // 16), _zrow, None)

    def _zdeg(t, carry):
        zdeg[pl.ds(t * 16, 16)] = zero16
        return carry
    lax.fori_loop(0, DPT // 16, _zdeg, None)

    def _ones(t, carry):
        ones_v[pl.ds(t * 16, 16)] = one16
        return carry
    lax.fori_loop(0, K // 16, _ones, None)

    for k in range(RPT // K):
        pltpu.sync_copy(rows0, acc.at[pl.ds(s * RPT + k * K, K)])

    if with_deg:
        @pl.when(c == 0)
        def _():
            pltpu.sync_copy(zdeg, degacc.at[pl.ds(s * DPT, DPT)])

    plsc.subcore_barrier()

    sbufs = (src0, src1)
    dbufs = (dst0, dst1)

    pltpu.sync_copy(src_hbm.at[c, s, 0], src0)
    pltpu.sync_copy(dst_hbm.at[s, 0], dst0)

    for p in range(PHASES):
        sv = sbufs[p % 2]
        dv = dbufs[p % 2]
        if p + 1 < PHASES:
            pltpu.async_copy(src_hbm.at[c, s, p + 1], sbufs[(p + 1) % 2], semi)
            pltpu.async_copy(dst_hbm.at[s, p + 1], dbufs[(p + 1) % 2], semi)

        def _start_gather(j, buf, sem, sv=sv):
            pltpu.async_copy(x_hbm.at[sv.at[j]], buf, sem)

        def _finish(j, buf, sem, sv=sv, dv=dv):
            pltpu.make_async_copy(x_hbm.at[sv.at[j]], buf, sem).wait()
            pltpu.sync_copy(buf, acc.at[dv.at[j]], add=True)

            if with_deg:
                @pl.when(c == 0)
                def _():
                    pltpu.sync_copy(ones_v, degacc.at[dv.at[j]], add=True)

        _start_gather(0, rows0, sem0)

        def _pair(t, carry):
            j0 = 2 * t
            _start_gather(j0 + 1, rows1, sem1)
            _finish(j0, rows0, sem0)

            @pl.when(j0 + 2 < CPP)
            def _():
                _start_gather(j0 + 2, rows0, sem0)
            _finish(j0 + 1, rows1, sem1)
            return carry
        lax.fori_loop(0, CPP // 2, _pair, None)
        _finish(CPP - 1, rows0, sem0)

        if p + 1 < PHASES:
            pltpu.make_async_copy(src_hbm.at[c, s, p + 1],
                                  sbufs[(p + 1) % 2], semi).wait()
            pltpu.make_async_copy(dst_hbm.at[s, p + 1],
                                  dbufs[(p + 1) % 2], semi).wait()

    plsc.subcore_barrier()

    pltpu.sync_copy(acc.at[pl.ds(s * RPT, RPT)],
                    summed_hbm.at[c, pl.ds(s * RPT, RPT)])

    if with_deg:
        @pl.when(c == 0)
        def _():
            pltpu.sync_copy(degacc.at[pl.ds(s * DPT, DPT)],
                            deg_hbm.at[pl.ds(s * DPT, DPT)])


def _make_sc_aggregate(with_deg):
    return functools.partial(
        pl.kernel,
        out_type=[jax.ShapeDtypeStruct((NC, NPAD, HD), jnp.float32),
                  jax.ShapeDtypeStruct((NPAD,), jnp.float32)],
        mesh=plsc.VectorSubcoreMesh(core_axis_name="c", subcore_axis_name="s"),
        scratch_types=[
            pltpu.VMEM((CPP, K), jnp.int32),
            pltpu.VMEM((CPP, K), jnp.int32),
            pltpu.VMEM((CPP, K), jnp.int32),
            pltpu.VMEM((CPP, K), jnp.int32),
            pltpu.VMEM((K, HD), jnp.float32),
            pltpu.VMEM((K, HD), jnp.float32),
            pltpu.VMEM((K,), jnp.float32),
            pltpu.VMEM((DPT,), jnp.float32),
            pltpu.VMEM_SHARED((NPAD, HD), jnp.float32),
            pltpu.VMEM_SHARED((NPAD,), jnp.float32),
            pltpu.SemaphoreType.DMA,
            pltpu.SemaphoreType.DMA,
            pltpu.SemaphoreType.DMA,
        ],
    )(functools.partial(_sc_agg_body, with_deg))


_sc_aggregate_deg = _make_sc_aggregate(True)
_sc_aggregate = _make_sc_aggregate(False)


def _tc_layer_body(relu, in_split, out_split,
                   sref, dref, xref, wlref, bref, wrref, oref):
    agg = jnp.concatenate([sref[0], sref[1]], axis=-1)
    rec = 1.0 / jnp.maximum(dref[...], 1.0)
    agg = agg * rec
    if in_split:
        xx = jnp.concatenate([xref[0], xref[1]], axis=-1)
    else:
        xx = xref[...]
    o = (jnp.dot(agg, wlref[...], preferred_element_type=jnp.float32)
         + bref[...]
         + jnp.dot(xx, wrref[...], preferred_element_type=jnp.float32))
    if relu:
        o = jnp.maximum(o, 0.0)
    if out_split:
        oref[0] = o[:, :HD]
        oref[1] = o[:, HD:]
    else:
        oref[...] = o


def _tc_layer(summed, deg_col, xin, W_l, b, W_r, *, relu, in_split, out_split):
    BM = 1000
    grid = (N // BM,)
    split_spec = pl.BlockSpec((NC, BM, HD), lambda i: (0, i, 0))
    dense_spec = pl.BlockSpec((BM, D), lambda i: (i, 0))
    in_specs = [
        split_spec,
        pl.BlockSpec((BM, 1), lambda i: (i, 0)),
        split_spec if in_split else dense_spec,
        pl.BlockSpec((D, D), lambda i: (0, 0)),
        pl.BlockSpec((1, D), lambda i: (0, 0)),
        pl.BlockSpec((D, D), lambda i: (0, 0)),
    ]
    if out_split:
        out_spec = split_spec
        out_shape = jax.ShapeDtypeStruct((NC, N, HD), jnp.float32)
    else:
        out_spec = dense_spec
        out_shape = jax.ShapeDtypeStruct((N, D), jnp.float32)
    return pl.pallas_call(
        functools.partial(_tc_layer_body, relu, in_split, out_split),
        grid=grid,
        in_specs=in_specs,
        out_specs=out_spec,
        out_shape=out_shape,
    )(summed, deg_col, xin, W_l, b.reshape(1, D), W_r)


def kernel(x, edge_index, W1_l, b1, W1_r, W2_l, b2, W2_r):
    src = edge_index[0].astype(jnp.int32)
    dst = edge_index[1].astype(jnp.int32)
    sr = src.reshape(NS, PHASES, CPP, K)
    src_idx = jnp.stack([sr, sr + N])
    dst_idx = dst.reshape(NS, PHASES, CPP, K)
    x_flat = jnp.concatenate([x[:, :HD], x[:, HD:]], axis=0)

    summed1, deg_pad = _sc_aggregate_deg(x_flat, src_idx, dst_idx)
    deg_col = deg_pad[:N].reshape(N, 1)
    h_split = _tc_layer(summed1, deg_col, x, W1_l, b1, W1_r,
                        relu=True, in_split=False, out_split=True)
    summed2, _ = _sc_aggregate(h_split.reshape(NC * N, HD), src_idx, dst_idx)
    out = _tc_layer(summed2, deg_col, h_split, W2_l, b2, W2_r,
                    relu=False, in_split=True, out_split=False)
    return out


# speedup vs baseline: 3.4902x; 1.1570x over previous
import functools

import jax
import jax.numpy as jnp
from jax import lax
from jax.experimental import pallas as pl
from jax.experimental.pallas import tpu as pltpu
from jax.experimental.pallas import tpu_sc as plsc

N = 10000
E = 160000
D = 256
HD = 128
NC = 2
NS = 16
EPT = E // NS
K = 80
NCHUNK = EPT // K
PHASES = 5
CPP = NCHUNK // PHASES
NPAD = 10240
RPT = NPAD // NS
ZROWS = 32
DPT = NPAD // NS


def _sc_agg_body(with_deg, x_hbm, src_hbm, dst_hbm, summed_hbm, deg_hbm,
                 src0, src1, dst0, dst1, rows0, rows1, rows2, ones_v, zdeg,
                 acc, degacc, sem0, sem1, sem2, semi):
    c = lax.axis_index("c")
    s = lax.axis_index("s")
    zero16 = jnp.zeros((16,), jnp.float32)
    one16 = jnp.ones((16,), jnp.float32)

    def _zrow(t, carry):
        i = t // (HD // 16)
        k = t % (HD // 16)
        rows0[i, pl.ds(k * 16, 16)] = zero16
        return carry
    lax.fori_loop(0, K * (HD // 16), _zrow, None)

    def _zdeg(t, carry):
        zdeg[pl.ds(t * 16, 16)] = zero16
        return carry
    lax.fori_loop(0, DPT // 16, _zdeg, None)

    def _ones(t, carry):
        ones_v[pl.ds(t * 16, 16)] = one16
        return carry
    lax.fori_loop(0, K // 16, _ones, None)

    for k in range(RPT // K):
        pltpu.sync_copy(rows0, acc.at[pl.ds(s * RPT + k * K, K)])

    if with_deg:
        @pl.when(c == 0)
        def _():
            pltpu.sync_copy(zdeg, degacc.at[pl.ds(s * DPT, DPT)])

    plsc.subcore_barrier()

    sbufs = (src0, src1)
    dbufs = (dst0, dst1)

    pltpu.sync_copy(src_hbm.at[c, s, 0], src0)
    pltpu.sync_copy(dst_hbm.at[s, 0], dst0)

    for p in range(PHASES):
        sv = sbufs[p % 2]
        dv = dbufs[p % 2]
        if p + 1 < PHASES:
            pltpu.async_copy(src_hbm.at[c, s, p + 1], sbufs[(p + 1) % 2], semi)
            pltpu.async_copy(dst_hbm.at[s, p + 1], dbufs[(p + 1) % 2], semi)

        bufs = ((rows0, sem0), (rows1, sem1), (rows2, sem2))

        def _start_gather(j, buf, sem, sv=sv):
            pltpu.async_copy(x_hbm.at[sv.at[j]], buf, sem)

        def _finish(j, buf, sem, sv=sv, dv=dv):
            pltpu.make_async_copy(x_hbm.at[sv.at[j]], buf, sem).wait()
            pltpu.sync_copy(buf, acc.at[dv.at[j]], add=True)

            if with_deg:
                @pl.when(c == 0)
                def _():
                    pltpu.sync_copy(ones_v, degacc.at[dv.at[j]], add=True)

        _start_gather(0, rows0, sem0)
        _start_gather(1, rows1, sem1)

        def _trip(t, carry):
            j0 = 3 * t
            for u in range(3):
                j = j0 + u
                bn, sn = bufs[(u + 2) % 3]

                @pl.when(j + 2 < CPP)
                def _(j=j, bn=bn, sn=sn):
                    _start_gather(j + 2, bn, sn)
                _finish(j, *bufs[u])
            return carry
        lax.fori_loop(0, CPP // 3, _trip, None)
        _finish(CPP - 1, *bufs[0])

        if p + 1 < PHASES:
            pltpu.make_async_copy(src_hbm.at[c, s, p + 1],
                                  sbufs[(p + 1) % 2], semi).wait()
            pltpu.make_async_copy(dst_hbm.at[s, p + 1],
                                  dbufs[(p + 1) % 2], semi).wait()

    plsc.subcore_barrier()

    pltpu.sync_copy(acc.at[pl.ds(s * RPT, RPT)],
                    summed_hbm.at[c, pl.ds(s * RPT, RPT)])

    if with_deg:
        @pl.when(c == 0)
        def _():
            pltpu.sync_copy(degacc.at[pl.ds(s * DPT, DPT)],
                            deg_hbm.at[pl.ds(s * DPT, DPT)])


def _make_sc_aggregate(with_deg):
    return functools.partial(
        pl.kernel,
        out_type=[jax.ShapeDtypeStruct((NC, NPAD, HD), jnp.float32),
                  jax.ShapeDtypeStruct((NPAD,), jnp.float32)],
        mesh=plsc.VectorSubcoreMesh(core_axis_name="c", subcore_axis_name="s"),
        scratch_types=[
            pltpu.VMEM((CPP, K), jnp.int32),
            pltpu.VMEM((CPP, K), jnp.int32),
            pltpu.VMEM((CPP, K), jnp.int32),
            pltpu.VMEM((CPP, K), jnp.int32),
            pltpu.VMEM((K, HD), jnp.float32),
            pltpu.VMEM((K, HD), jnp.float32),
            pltpu.VMEM((K, HD), jnp.float32),
            pltpu.VMEM((K,), jnp.float32),
            pltpu.VMEM((DPT,), jnp.float32),
            pltpu.VMEM_SHARED((NPAD, HD), jnp.float32),
            pltpu.VMEM_SHARED((NPAD,), jnp.float32),
            pltpu.SemaphoreType.DMA,
            pltpu.SemaphoreType.DMA,
            pltpu.SemaphoreType.DMA,
            pltpu.SemaphoreType.DMA,
        ],
    )(functools.partial(_sc_agg_body, with_deg))


_sc_aggregate_deg = _make_sc_aggregate(True)
_sc_aggregate = _make_sc_aggregate(False)


def _tc_layer_body(relu, in_split, out_split,
                   sref, dref, xref, wlref, bref, wrref, oref):
    agg = jnp.concatenate([sref[0], sref[1]], axis=-1)
    rec = 1.0 / jnp.maximum(dref[...], 1.0)
    agg = agg * rec
    if in_split:
        xx = jnp.concatenate([xref[0], xref[1]], axis=-1)
    else:
        xx = xref[...]
    o = (jnp.dot(agg, wlref[...], preferred_element_type=jnp.float32)
         + bref[...]
         + jnp.dot(xx, wrref[...], preferred_element_type=jnp.float32))
    if relu:
        o = jnp.maximum(o, 0.0)
    if out_split:
        oref[0] = o[:, :HD]
        oref[1] = o[:, HD:]
    else:
        oref[...] = o


def _tc_layer(summed, deg_col, xin, W_l, b, W_r, *, relu, in_split, out_split):
    BM = 2000
    grid = (N // BM,)
    split_spec = pl.BlockSpec((NC, BM, HD), lambda i: (0, i, 0))
    dense_spec = pl.BlockSpec((BM, D), lambda i: (i, 0))
    in_specs = [
        split_spec,
        pl.BlockSpec((BM, 1), lambda i: (i, 0)),
        split_spec if in_split else dense_spec,
        pl.BlockSpec((D, D), lambda i: (0, 0)),
        pl.BlockSpec((1, D), lambda i: (0, 0)),
        pl.BlockSpec((D, D), lambda i: (0, 0)),
    ]
    if out_split:
        out_spec = split_spec
        out_shape = jax.ShapeDtypeStruct((NC, N, HD), jnp.float32)
    else:
        out_spec = dense_spec
        out_shape = jax.ShapeDtypeStruct((N, D), jnp.float32)
    return pl.pallas_call(
        functools.partial(_tc_layer_body, relu, in_split, out_split),
        grid=grid,
        in_specs=in_specs,
        out_specs=out_spec,
        out_shape=out_shape,
    )(summed, deg_col, xin, W_l, b.reshape(1, D), W_r)


def kernel(x, edge_index, W1_l, b1, W1_r, W2_l, b2, W2_r):
    src = edge_index[0].astype(jnp.int32)
    dst = edge_index[1].astype(jnp.int32)
    sr = src.reshape(NS, PHASES, CPP, K)
    src_idx = jnp.stack([sr, sr + N])
    dst_idx = dst.reshape(NS, PHASES, CPP, K)
    x_flat = jnp.concatenate([x[:, :HD], x[:, HD:]], axis=0)

    summed1, deg_pad = _sc_aggregate_deg(x_flat, src_idx, dst_idx)
    deg_col = deg_pad[:N].reshape(N, 1)
    h_split = _tc_layer(summed1, deg_col, x, W1_l, b1, W1_r,
                        relu=True, in_split=False, out_split=True)
    summed2, _ = _sc_aggregate(h_split.reshape(NC * N, HD), src_idx, dst_idx)
    out = _tc_layer(summed2, deg_col, h_split, W2_l, b2, W2_r,
                    relu=False, in_split=True, out_split=False)
    return out
